# baseline (speedup 1.0000x reference)
import jax
import jax.numpy as jnp
from jax import lax
from jax.experimental import pallas as pl
from jax.experimental.pallas import tpu as pltpu
from jax.experimental.pallas import tpu_sc as plsc

N = 10000
D = 128
NUM_TYPES = 64
E = 320000
P = 100000

NC = 2
NS = 16
NW = NC * NS
CK = 128

EPW = E // NW
FC = EPW // CK
TAIL = EPW - FC * CK

PC = -(-P // (NW * CK))
P_PAD = NW * CK * PC
NPAD = 10240
RT = NPAD // NS

BN = 2000
BE = 2000

_MESH = plsc.VectorSubcoreMesh(
    core_axis_name="c", subcore_axis_name="s", num_cores=NC, num_subcores=NS)


def _dot(a, b):
    return jnp.dot(a, b, preferred_element_type=jnp.float32)


def _embed_body(seq_ref, emb_ref, o_ref):
    s = seq_ref[...]
    oh = (s == lax.broadcasted_iota(jnp.int32, (BN, NUM_TYPES), 1))
    o_ref[...] = jnp.dot(oh.astype(jnp.float32), emb_ref[...],
                         preferred_element_type=jnp.float32,
                         precision=lax.Precision.HIGHEST)


def _embed(seq_col, emb_table):
    return pl.pallas_call(
        _embed_body,
        grid=(N // BN,),
        in_specs=[
            pl.BlockSpec((BN, 1), lambda i: (i, 0)),
            pl.BlockSpec((NUM_TYPES, D), lambda i: (0, 0)),
        ],
        out_specs=pl.BlockSpec((BN, D), lambda i: (i, 0)),
        out_shape=jax.ShapeDtypeStruct((N, D), jnp.float32),
    )(seq_col, emb_table)


def _gather_body(x_hbm, src_hbm, xg_hbm, sidx, sidx_t, rows, rows_t, sem):
    c = lax.axis_index("c")
    s = lax.axis_index("s")
    wid = c * NS + s
    base = wid * EPW

    def chunk(i, carry):
        off = base + i * CK
        pltpu.sync_copy(src_hbm.at[pl.ds(off, CK)], sidx)
        pltpu.async_copy(x_hbm.at[sidx], rows, sem).wait()
        pltpu.sync_copy(rows, xg_hbm.at[pl.ds(off, CK)])
        return carry

    lax.fori_loop(0, FC, chunk, 0)
    off = base + FC * CK
    pltpu.sync_copy(src_hbm.at[pl.ds(off, TAIL)], sidx_t)
    pltpu.async_copy(x_hbm.at[sidx_t], rows_t, sem).wait()
    pltpu.sync_copy(rows_t, xg_hbm.at[pl.ds(off, TAIL)])


_gather = pl.kernel(
    _gather_body,
    out_type=jax.ShapeDtypeStruct((E, D), jnp.float32),
    mesh=_MESH,
    scratch_types=[
        pltpu.VMEM((CK,), jnp.int32),
        pltpu.VMEM((TAIL,), jnp.int32),
        pltpu.VMEM((CK, D), jnp.float32),
        pltpu.VMEM((TAIL, D), jnp.float32),
        pltpu.SemaphoreType.DMA,
    ],
)


def _msg_body(xg_ref, ea_ref, wm_ref, bm_ref, o_ref):
    a = jnp.concatenate([xg_ref[...], ea_ref[...]], axis=1)
    o_ref[...] = _dot(a, wm_ref[...]) + bm_ref[...]


def _msg(xg, ea, wm, bm_row):
    return pl.pallas_call(
        _msg_body,
        grid=(E // BE,),
        in_specs=[
            pl.BlockSpec((BE, D), lambda i: (i, 0)),
            pl.BlockSpec((BE, 1), lambda i: (i, 0)),
            pl.BlockSpec((D + 1, D), lambda i: (0, 0)),
            pl.BlockSpec((1, D), lambda i: (0, 0)),
        ],
        out_specs=pl.BlockSpec((BE, D), lambda i: (i, 0)),
        out_shape=jax.ShapeDtypeStruct((E, D), jnp.float32),
    )(xg, ea, wm, bm_row)


def _scatter_body(msg_hbm, dst_hbm, zr_hbm, xp_hbm,
                  didx, didx_t, rows, rows_t, acc):
    c = lax.axis_index("c")
    s = lax.axis_index("s")
    wid = c * NS + s
    pltpu.sync_copy(zr_hbm, acc.at[pl.ds(s * RT, RT)])
    plsc.subcore_barrier()

    base = wid * EPW

    def chunk(i, carry):
        off = base + i * CK
        pltpu.sync_copy(dst_hbm.at[pl.ds(off, CK)], didx)
        pltpu.sync_copy(msg_hbm.at[pl.ds(off, CK)], rows)
        pltpu.sync_copy(rows, acc.at[didx], add=True)
        return carry

    lax.fori_loop(0, FC, chunk, 0)
    off = base + FC * CK
    pltpu.sync_copy(dst_hbm.at[pl.ds(off, TAIL)], didx_t)
    pltpu.sync_copy(msg_hbm.at[pl.ds(off, TAIL)], rows_t)
    pltpu.sync_copy(rows_t, acc.at[didx_t], add=True)

    plsc.subcore_barrier()
    pltpu.sync_copy(acc.at[pl.ds(s * RT, RT)], xp_hbm.at[c, pl.ds(s * RT, RT)])


_scatter = pl.kernel(
    _scatter_body,
    out_type=jax.ShapeDtypeStruct((NC, NPAD, D), jnp.float32),
    mesh=_MESH,
    scratch_types=[
        pltpu.VMEM((CK,), jnp.int32),
        pltpu.VMEM((TAIL,), jnp.int32),
        pltpu.VMEM((CK, D), jnp.float32),
        pltpu.VMEM((TAIL, D), jnp.float32),
        pltpu.VMEM_SHARED((NPAD, D), jnp.float32),
    ],
)


def _dense_body(x_ref, xp_ref, ws_ref, bs_ref, o_ref):
    agg = xp_ref[0] + xp_ref[1]
    h = (_dot(x_ref[...], ws_ref[...]) + bs_ref[...]) + agg
    o_ref[...] = jnp.maximum(h, 0.0)


def _dense_layer(x, xp, ws, bs_row):
    return pl.pallas_call(
        _dense_body,
        grid=(N // BN,),
        in_specs=[
            pl.BlockSpec((BN, D), lambda i: (i, 0)),
            pl.BlockSpec((NC, BN, D), lambda i: (0, i, 0)),
            pl.BlockSpec((D, D), lambda i: (0, 0)),
            pl.BlockSpec((1, D), lambda i: (0, 0)),
        ],
        out_specs=pl.BlockSpec((BN, D), lambda i: (i, 0)),
        out_shape=jax.ShapeDtypeStruct((N, D), jnp.float32),
    )(x, xp, ws, bs_row)


def _heads_body(x_ref, wo_ref, bo_ref, wh_ref, bh_ref, o_ref):
    emb = _dot(x_ref[...], wo_ref[...]) + bo_ref[...]
    h = _dot(emb, wh_ref[...]) + bh_ref[...]
    col = lax.broadcasted_iota(jnp.int32, h.shape, 1)
    h = jnp.where(col == 2, jax.nn.sigmoid(h), h)
    o_ref[...] = h[:, :8]


def _heads(x, w_out, bo_row, wh, bh_row):
    return pl.pallas_call(
        _heads_body,
        grid=(N // BN,),
        in_specs=[
            pl.BlockSpec((BN, D), lambda i: (i, 0)),
            pl.BlockSpec((D, D), lambda i: (0, 0)),
            pl.BlockSpec((1, D), lambda i: (0, 0)),
            pl.BlockSpec((D, D), lambda i: (0, 0)),
            pl.BlockSpec((1, D), lambda i: (0, 0)),
        ],
        out_specs=pl.BlockSpec((BN, 8), lambda i: (i, 0)),
        out_shape=jax.ShapeDtypeStruct((N, 8), jnp.float32),
    )(x, w_out, bo_row, wh, bh_row)


def _edge_head_body(le_hbm, re_hbm, p0_hbm, p1_hbm, o_hbm,
                    i0, i1, a_v, b_v, o_v, sem):
    c = lax.axis_index("c")
    s = lax.axis_index("s")
    wid = c * NS + s
    base = wid * (PC * CK)

    def chunk(i, carry):
        off = base + i * CK
        pltpu.sync_copy(p0_hbm.at[pl.ds(off, CK)], i0)
        pltpu.sync_copy(p1_hbm.at[pl.ds(off, CK)], i1)
        pltpu.async_copy(le_hbm.at[i0], a_v, sem).wait()
        pltpu.async_copy(re_hbm.at[i1], b_v, sem).wait()
        for j in range(CK // 16):
            sl = pl.ds(j * 16, 16)
            z = a_v[sl] + b_v[sl]
            o_v[sl] = 1.0 / (1.0 + jnp.exp(-z))
        pltpu.sync_copy(o_v, o_hbm.at[pl.ds(off, CK)])
        return carry

    lax.fori_loop(0, PC, chunk, 0)


_edge_head = pl.kernel(
    _edge_head_body,
    out_type=jax.ShapeDtypeStruct((P_PAD,), jnp.float32),
    mesh=_MESH,
    scratch_types=[
        pltpu.VMEM((CK,), jnp.int32),
        pltpu.VMEM((CK,), jnp.int32),
        pltpu.VMEM((CK,), jnp.float32),
        pltpu.VMEM((CK,), jnp.float32),
        pltpu.VMEM((CK,), jnp.float32),
        pltpu.SemaphoreType.DMA,
    ],
)


def kernel(seq, nedge, edgeattr, pedge, emb_table,
           Wm1, bm1, Ws1, bs1, Wm2, bm2, Ws2, bs2,
           W_out, b_out, W_e, b_e, W_n, b_n):
    f32 = jnp.float32
    seq_col = seq.astype(jnp.int32).reshape(N, 1)
    src = nedge[0].astype(jnp.int32)
    dst = nedge[1].astype(jnp.int32)
    ea = edgeattr.astype(f32)
    ppad = P_PAD - P
    p0p = jnp.concatenate([pedge[0].astype(jnp.int32),
                           jnp.zeros((ppad,), jnp.int32)])
    p1p = jnp.concatenate([pedge[1].astype(jnp.int32),
                           jnp.zeros((ppad,), jnp.int32)])
    zr = jnp.zeros((RT, D), f32)

    x0 = _embed(seq_col, emb_table)

    xg1 = _gather(x0, src)
    msg1 = _msg(xg1, ea, Wm1, bm1[None, :])
    xp1 = _scatter(msg1, dst, zr)
    x1 = _dense_layer(x0, xp1, Ws1, bs1[None, :])

    xg2 = _gather(x1, src)
    msg2 = _msg(xg2, ea, Wm2, bm2[None, :])
    xp2 = _scatter(msg2, dst, zr)
    x2 = _dense_layer(x1, xp2, Ws2, bs2[None, :])

    wh = jnp.concatenate([W_e[:D], W_e[D:], W_n, jnp.zeros((D, 125), f32)],
                         axis=1)
    bh = jnp.concatenate([b_e, jnp.zeros((1,), f32), b_n,
                          jnp.zeros((125,), f32)])[None, :]
    lrp = _heads(x2, W_out, b_out[None, :], wh, bh)

    le = lrp[:, 0]
    re = lrp[:, 1]
    pred_node = lrp[:, 2:3]
    pe = _edge_head(le, re, p0p, p1p)
    pred_edge = pe[:P].reshape(P, 1)
    return (pred_edge, pred_node)


# speedup vs baseline: 1.2499x; 1.2499x over previous
import jax
import jax.numpy as jnp
from jax import lax
from jax.experimental import pallas as pl
from jax.experimental.pallas import tpu as pltpu
from jax.experimental.pallas import tpu_sc as plsc

N = 10000
D = 128
NUM_TYPES = 64
E = 320000
P = 100000

NC = 2
NS = 16
NW = NC * NS
CK = 128

EPW = E // NW
FC = EPW // CK
TAIL = EPW - FC * CK

PC = -(-P // (NW * CK))
P_PAD = NW * CK * PC
NPAD = 10240
RT = NPAD // NS

BN = 2000
BE = 2000

_MESH = plsc.VectorSubcoreMesh(
    core_axis_name="c", subcore_axis_name="s", num_cores=NC, num_subcores=NS)


def _dot(a, b):
    return jnp.dot(a, b, preferred_element_type=jnp.float32)


def _embed_body(seq_ref, emb_ref, o_ref):
    s = seq_ref[...]
    oh = (s == lax.broadcasted_iota(jnp.int32, (BN, NUM_TYPES), 1))
    o_ref[...] = jnp.dot(oh.astype(jnp.float32), emb_ref[...],
                         preferred_element_type=jnp.float32,
                         precision=lax.Precision.HIGHEST)


def _embed(seq_col, emb_table):
    return pl.pallas_call(
        _embed_body,
        grid=(N // BN,),
        in_specs=[
            pl.BlockSpec((BN, 1), lambda i: (i, 0)),
            pl.BlockSpec((NUM_TYPES, D), lambda i: (0, 0)),
        ],
        out_specs=pl.BlockSpec((BN, D), lambda i: (i, 0)),
        out_shape=jax.ShapeDtypeStruct((N, D), jnp.float32),
    )(seq_col, emb_table)


def _gather_body(x_hbm, src_hbm, xg_hbm, sidx_all, sidx_t,
                 rows0, rows1, rows_t, gsem0, gsem1, wsem0, wsem1, sem_t):
    c = lax.axis_index("c")
    s = lax.axis_index("s")
    wid = c * NS + s
    base = wid * EPW
    pltpu.sync_copy(src_hbm.at[pl.ds(base, FC * CK)], sidx_all)

    def idx(i):
        return sidx_all.at[pl.ds(i * CK, CK)]

    def start_g(i, rows, sem):
        pltpu.async_copy(x_hbm.at[idx(i)], rows, sem)

    def wait_g(i, rows, sem):
        pltpu.make_async_copy(x_hbm.at[idx(i)], rows, sem).wait()

    def start_w(i, rows, sem):
        pltpu.async_copy(rows, xg_hbm.at[pl.ds(base + i * CK, CK)], sem)

    def wait_w(i, rows, sem):
        pltpu.make_async_copy(rows, xg_hbm.at[pl.ds(base + i * CK, CK)],
                              sem).wait()

    start_g(0, rows0, gsem0)
    start_g(1, rows1, gsem1)

    def pair(j, carry):
        i0 = 2 * j
        wait_g(i0, rows0, gsem0)
        start_w(i0, rows0, wsem0)
        wait_g(i0 + 1, rows1, gsem1)
        start_w(i0 + 1, rows1, wsem1)

        @pl.when(j < FC // 2 - 1)
        def _():
            wait_w(i0, rows0, wsem0)
            start_g(i0 + 2, rows0, gsem0)
            wait_w(i0 + 1, rows1, wsem1)
            start_g(i0 + 3, rows1, gsem1)

        return carry

    lax.fori_loop(0, FC // 2, pair, 0)
    wait_w(FC - 2, rows0, wsem0)
    wait_w(FC - 1, rows1, wsem1)

    off = base + FC * CK
    pltpu.sync_copy(src_hbm.at[pl.ds(off, TAIL)], sidx_t)
    pltpu.async_copy(x_hbm.at[sidx_t], rows_t, sem_t).wait()
    pltpu.sync_copy(rows_t, xg_hbm.at[pl.ds(off, TAIL)])


_gather = pl.kernel(
    _gather_body,
    out_type=jax.ShapeDtypeStruct((E, D), jnp.float32),
    mesh=_MESH,
    scratch_types=[
        pltpu.VMEM((FC * CK,), jnp.int32),
        pltpu.VMEM((TAIL,), jnp.int32),
        pltpu.VMEM((CK, D), jnp.float32),
        pltpu.VMEM((CK, D), jnp.float32),
        pltpu.VMEM((TAIL, D), jnp.float32),
        pltpu.SemaphoreType.DMA,
        pltpu.SemaphoreType.DMA,
        pltpu.SemaphoreType.DMA,
        pltpu.SemaphoreType.DMA,
        pltpu.SemaphoreType.DMA,
    ],
)


def _msg_body(xg_ref, ea_ref, wm_ref, bm_ref, o_ref):
    a = jnp.concatenate([xg_ref[...], ea_ref[...]], axis=1)
    o_ref[...] = _dot(a, wm_ref[...]) + bm_ref[...]


def _msg(xg, ea, wm, bm_row):
    return pl.pallas_call(
        _msg_body,
        grid=(E // BE,),
        in_specs=[
            pl.BlockSpec((BE, D), lambda i: (i, 0)),
            pl.BlockSpec((BE, 1), lambda i: (i, 0)),
            pl.BlockSpec((D + 1, D), lambda i: (0, 0)),
            pl.BlockSpec((1, D), lambda i: (0, 0)),
        ],
        out_specs=pl.BlockSpec((BE, D), lambda i: (i, 0)),
        out_shape=jax.ShapeDtypeStruct((E, D), jnp.float32),
    )(xg, ea, wm, bm_row)


def _scatter_body(msg_hbm, dst_hbm, zr_hbm, xp_hbm,
                  didx0, didx1, didx_t, rows0, rows1, rows_t,
                  rsem0, rsem1, ssem0, ssem1, acc):
    c = lax.axis_index("c")
    s = lax.axis_index("s")
    wid = c * NS + s
    pltpu.sync_copy(zr_hbm, acc.at[pl.ds(s * RT, RT)])
    plsc.subcore_barrier()

    base = wid * EPW

    def load(i, didx, rows, sem):
        pltpu.sync_copy(dst_hbm.at[pl.ds(base + i * CK, CK)], didx)
        pltpu.async_copy(msg_hbm.at[pl.ds(base + i * CK, CK)], rows, sem)

    def wait_load(i, rows, sem):
        pltpu.make_async_copy(msg_hbm.at[pl.ds(base + i * CK, CK)], rows,
                              sem).wait()

    def start_sa(didx, rows, sem):
        pltpu.async_copy(rows, acc.at[didx], sem, add=True)

    def wait_sa(didx, rows, sem):
        pltpu.make_async_copy(rows, acc.at[didx], sem).wait()

    load(0, didx0, rows0, rsem0)
    load(1, didx1, rows1, rsem1)

    def pair(j, carry):
        i0 = 2 * j
        wait_load(i0, rows0, rsem0)
        start_sa(didx0, rows0, ssem0)
        wait_load(i0 + 1, rows1, rsem1)
        start_sa(didx1, rows1, ssem1)

        @pl.when(j < FC // 2 - 1)
        def _():
            wait_sa(didx0, rows0, ssem0)
            load(i0 + 2, didx0, rows0, rsem0)
            wait_sa(didx1, rows1, ssem1)
            load(i0 + 3, didx1, rows1, rsem1)

        return carry

    lax.fori_loop(0, FC // 2, pair, 0)
    wait_sa(didx0, rows0, ssem0)
    wait_sa(didx1, rows1, ssem1)

    off = base + FC * CK
    pltpu.sync_copy(dst_hbm.at[pl.ds(off, TAIL)], didx_t)
    pltpu.sync_copy(msg_hbm.at[pl.ds(off, TAIL)], rows_t)
    pltpu.sync_copy(rows_t, acc.at[didx_t], add=True)

    plsc.subcore_barrier()
    pltpu.sync_copy(acc.at[pl.ds(s * RT, RT)], xp_hbm.at[c, pl.ds(s * RT, RT)])


_scatter = pl.kernel(
    _scatter_body,
    out_type=jax.ShapeDtypeStruct((NC, NPAD, D), jnp.float32),
    mesh=_MESH,
    scratch_types=[
        pltpu.VMEM((CK,), jnp.int32),
        pltpu.VMEM((CK,), jnp.int32),
        pltpu.VMEM((TAIL,), jnp.int32),
        pltpu.VMEM((CK, D), jnp.float32),
        pltpu.VMEM((CK, D), jnp.float32),
        pltpu.VMEM((TAIL, D), jnp.float32),
        pltpu.SemaphoreType.DMA,
        pltpu.SemaphoreType.DMA,
        pltpu.SemaphoreType.DMA,
        pltpu.SemaphoreType.DMA,
        pltpu.VMEM_SHARED((NPAD, D), jnp.float32),
    ],
)


def _dense_body(x_ref, xp_ref, ws_ref, bs_ref, o_ref):
    agg = xp_ref[0] + xp_ref[1]
    h = (_dot(x_ref[...], ws_ref[...]) + bs_ref[...]) + agg
    o_ref[...] = jnp.maximum(h, 0.0)


def _dense_layer(x, xp, ws, bs_row):
    return pl.pallas_call(
        _dense_body,
        grid=(N // BN,),
        in_specs=[
            pl.BlockSpec((BN, D), lambda i: (i, 0)),
            pl.BlockSpec((NC, BN, D), lambda i: (0, i, 0)),
            pl.BlockSpec((D, D), lambda i: (0, 0)),
            pl.BlockSpec((1, D), lambda i: (0, 0)),
        ],
        out_specs=pl.BlockSpec((BN, D), lambda i: (i, 0)),
        out_shape=jax.ShapeDtypeStruct((N, D), jnp.float32),
    )(x, xp, ws, bs_row)


def _heads_body(x_ref, wo_ref, bo_ref, wh_ref, bh_ref, o_ref):
    emb = _dot(x_ref[...], wo_ref[...]) + bo_ref[...]
    h = _dot(emb, wh_ref[...]) + bh_ref[...]
    col = lax.broadcasted_iota(jnp.int32, h.shape, 1)
    h = jnp.where(col == 2, jax.nn.sigmoid(h), h)
    o_ref[...] = h[:, :8]


def _heads(x, w_out, bo_row, wh, bh_row):
    return pl.pallas_call(
        _heads_body,
        grid=(N // BN,),
        in_specs=[
            pl.BlockSpec((BN, D), lambda i: (i, 0)),
            pl.BlockSpec((D, D), lambda i: (0, 0)),
            pl.BlockSpec((1, D), lambda i: (0, 0)),
            pl.BlockSpec((D, D), lambda i: (0, 0)),
            pl.BlockSpec((1, D), lambda i: (0, 0)),
        ],
        out_specs=pl.BlockSpec((BN, 8), lambda i: (i, 0)),
        out_shape=jax.ShapeDtypeStruct((N, 8), jnp.float32),
    )(x, w_out, bo_row, wh, bh_row)


def _edge_head_body(le_hbm, re_hbm, p0_hbm, p1_hbm, o_hbm,
                    i0, i1, a_v, b_v, o_v, sem):
    c = lax.axis_index("c")
    s = lax.axis_index("s")
    wid = c * NS + s
    base = wid * (PC * CK)

    def chunk(i, carry):
        off = base + i * CK
        pltpu.sync_copy(p0_hbm.at[pl.ds(off, CK)], i0)
        pltpu.sync_copy(p1_hbm.at[pl.ds(off, CK)], i1)
        pltpu.async_copy(le_hbm.at[i0], a_v, sem).wait()
        pltpu.async_copy(re_hbm.at[i1], b_v, sem).wait()
        for j in range(CK // 16):
            sl = pl.ds(j * 16, 16)
            z = a_v[sl] + b_v[sl]
            o_v[sl] = 1.0 / (1.0 + jnp.exp(-z))
        pltpu.sync_copy(o_v, o_hbm.at[pl.ds(off, CK)])
        return carry

    lax.fori_loop(0, PC, chunk, 0)


_edge_head = pl.kernel(
    _edge_head_body,
    out_type=jax.ShapeDtypeStruct((P_PAD,), jnp.float32),
    mesh=_MESH,
    scratch_types=[
        pltpu.VMEM((CK,), jnp.int32),
        pltpu.VMEM((CK,), jnp.int32),
        pltpu.VMEM((CK,), jnp.float32),
        pltpu.VMEM((CK,), jnp.float32),
        pltpu.VMEM((CK,), jnp.float32),
        pltpu.SemaphoreType.DMA,
    ],
)


def kernel(seq, nedge, edgeattr, pedge, emb_table,
           Wm1, bm1, Ws1, bs1, Wm2, bm2, Ws2, bs2,
           W_out, b_out, W_e, b_e, W_n, b_n):
    f32 = jnp.float32
    seq_col = seq.astype(jnp.int32).reshape(N, 1)
    src = nedge[0].astype(jnp.int32)
    dst = nedge[1].astype(jnp.int32)
    ea = edgeattr.astype(f32)
    ppad = P_PAD - P
    p0p = jnp.concatenate([pedge[0].astype(jnp.int32),
                           jnp.zeros((ppad,), jnp.int32)])
    p1p = jnp.concatenate([pedge[1].astype(jnp.int32),
                           jnp.zeros((ppad,), jnp.int32)])
    zr = jnp.zeros((RT, D), f32)

    x0 = _embed(seq_col, emb_table)

    xg1 = _gather(x0, src)
    msg1 = _msg(xg1, ea, Wm1, bm1[None, :])
    xp1 = _scatter(msg1, dst, zr)
    x1 = _dense_layer(x0, xp1, Ws1, bs1[None, :])

    xg2 = _gather(x1, src)
    msg2 = _msg(xg2, ea, Wm2, bm2[None, :])
    xp2 = _scatter(msg2, dst, zr)
    x2 = _dense_layer(x1, xp2, Ws2, bs2[None, :])

    wh = jnp.concatenate([W_e[:D], W_e[D:], W_n, jnp.zeros((D, 125), f32)],
                         axis=1)
    bh = jnp.concatenate([b_e, jnp.zeros((1,), f32), b_n,
                          jnp.zeros((125,), f32)])[None, :]
    lrp = _heads(x2, W_out, b_out[None, :], wh, bh)

    le = lrp[:, 0]
    re = lrp[:, 1]
    pred_node = lrp[:, 2:3]
    pe = _edge_head(le, re, p0p, p1p)
    pred_edge = pe[:P].reshape(P, 1)
    return (pred_edge, pred_node)
